# baseline (speedup 1.0000x reference)
import functools

import jax
import jax.numpy as jnp
from jax import lax
from jax.experimental import pallas as pl
from jax.experimental.pallas import tpu as pltpu
from jax.experimental.pallas import tpu_sc as plsc

_NC = 2
_NS = 16
_NW = _NC * _NS
_L = 16
_CHUNK = 256


def _sc_body(n, d, x_hbm, r_hbm, m_hbm, t_hbm, out_hbm,
             xbuf, rbuf, mbuf, tbuf, sems):
    rows_per_w = n // _NW
    nchunk = rows_per_w // _CHUNK
    wid = lax.axis_index("s") * _NC + lax.axis_index("c")
    base_row = wid * rows_per_w

    t_cp = pltpu.async_copy(t_hbm, tbuf, sems[4])

    def start_in(g, slot):
        row0 = base_row + g * _CHUNK
        h1 = pltpu.async_copy(x_hbm.at[pl.ds(row0 * d, _CHUNK * d)],
                              xbuf.at[slot], sems[slot])
        h2 = pltpu.async_copy(r_hbm.at[pl.ds(row0, _CHUNK)],
                              rbuf.at[slot], sems[slot])
        h3 = pltpu.async_copy(m_hbm.at[pl.ds(row0, _CHUNK)],
                              mbuf.at[slot], sems[slot])
        return (h1, h2, h3)

    def start_out(g, slot):
        row0 = base_row + g * _CHUNK
        return pltpu.async_copy(xbuf.at[slot],
                                out_hbm.at[pl.ds(row0 * d, _CHUNK * d)],
                                sems[2 + slot])

    def compute(slot):
        def grp_body(g16, carry):
            r0 = g16 * _L
            rv = rbuf[slot, pl.ds(r0, _L)]
            mv = mbuf[slot, pl.ds(r0, _L)]
            toff = jnp.where(mv != 0, jnp.where(rv >= 94, 2 * d, d), 0)
            for k in range(_L):
                tk = toff[k]
                base = (r0 + k) * d
                for j in range(d // _L):
                    sl = pl.ds(base + j * _L, _L)
                    xbuf[slot, sl] = (xbuf[slot, sl]
                                      + tbuf[pl.ds(tk + j * _L, _L)])
            return carry
        lax.fori_loop(0, _CHUNK // _L, grp_body, 0, unroll=False)

    in_h = [None] * nchunk
    out_h = [None] * nchunk
    in_h[0] = start_in(0, 0)
    t_cp.wait()
    for g in range(nchunk):
        slot = g % 2
        if g + 1 < nchunk:
            if g >= 1:
                out_h[g - 1].wait()
            in_h[g + 1] = start_in(g + 1, 1 - slot)
        for h in in_h[g]:
            h.wait()
        compute(slot)
        out_h[g] = start_out(g, slot)
    out_h[nchunk - 1].wait()
    if nchunk >= 2:
        out_h[nchunk - 2].wait()


def kernel(x, resids, mask, table):
    n, d = x.shape
    t4 = jnp.concatenate([jnp.zeros((1, d), table.dtype), table], axis=0)
    r32 = resids.astype(jnp.int32)
    m32 = mask.astype(jnp.int32)

    mesh = plsc.VectorSubcoreMesh(core_axis_name="c", subcore_axis_name="s",
                                  num_cores=_NC, num_subcores=_NS)
    sc = functools.partial(
        pl.kernel,
        out_type=jax.ShapeDtypeStruct((n * d,), jnp.float32),
        mesh=mesh,
        scratch_types=[
            pltpu.VMEM((2, _CHUNK * d), jnp.float32),
            pltpu.VMEM((2, _CHUNK), jnp.int32),
            pltpu.VMEM((2, _CHUNK), jnp.int32),
            pltpu.VMEM((3 * d,), jnp.float32),
            [pltpu.SemaphoreType.DMA] * 5,
        ],
    )(functools.partial(_sc_body, n, d))
    out = sc(x.reshape(n * d), r32, m32, t4.reshape(3 * d))
    return out.reshape(n, d)
